# baseline (speedup 1.0000x reference)
import functools

import jax
import jax.numpy as jnp
from jax import lax
from jax.experimental import pallas as pl
from jax.experimental.pallas import tpu as pltpu
from jax.experimental.pallas import tpu_sc as plsc

EMBEDDING_DIM = 32
_NC = 2
_NS = 16
_NW = _NC * _NS
_NBUF = 8


@functools.lru_cache(maxsize=None)
def _make_gather(batch: int, hist: int, dim: int):
    b_per_w = batch // _NW
    assert b_per_w % _NBUF == 0
    mesh = plsc.VectorSubcoreMesh(core_axis_name="c", subcore_axis_name="s")

    scratch = (
        [pltpu.VMEM((hist,), jnp.int32) for _ in range(_NBUF)]
        + [pltpu.VMEM((hist, dim), jnp.float32) for _ in range(_NBUF)]
        + [pltpu.SemaphoreType.DMA for _ in range(3 * _NBUF)]
    )

    @functools.partial(
        pl.kernel,
        mesh=mesh,
        out_type=jax.ShapeDtypeStruct((batch, hist, dim), jnp.float32),
        scratch_types=scratch,
        compiler_params=pltpu.CompilerParams(use_tc_tiling_on_sc=False),
    )
    def gather_kernel(ids_hbm, table_hbm, out_hbm, *bufs):
        idx_v = bufs[0:_NBUF]
        rows_v = bufs[_NBUF:2 * _NBUF]
        sem_i = bufs[2 * _NBUF:3 * _NBUF]
        sem_r = bufs[3 * _NBUF:4 * _NBUF]
        sem_o = bufs[4 * _NBUF:5 * _NBUF]

        wid = lax.axis_index("s") * _NC + lax.axis_index("c")
        row_base = wid * b_per_w

        def ids_start(j, b):
            pltpu.async_copy(ids_hbm.at[row_base + j, :], idx_v[b], sem_i[b])

        def ids_wait(b):
            pltpu.make_async_copy(ids_hbm.at[0, :], idx_v[b], sem_i[b]).wait()

        def store_wait(b):
            pltpu.make_async_copy(
                rows_v[b], out_hbm.at[0, :, :], sem_o[b]
            ).wait()

        for b in range(_NBUF):
            ids_start(b, b)

        def group(g, carry):
            for b in range(_NBUF):
                ids_wait(b)

                @pl.when(g > 0)
                def _():
                    store_wait(b)

                pltpu.async_copy(table_hbm.at[idx_v[b]], rows_v[b], sem_r[b])

            for b in range(_NBUF):
                j = g * _NBUF + b
                pltpu.make_async_copy(
                    table_hbm.at[idx_v[b]], rows_v[b], sem_r[b]
                ).wait()

                pltpu.async_copy(
                    rows_v[b], out_hbm.at[row_base + j, :, :], sem_o[b]
                )

                @pl.when(j + _NBUF < b_per_w)
                def _():
                    ids_start(j + _NBUF, b)

            return carry

        lax.fori_loop(0, b_per_w // _NBUF, group, 0)

        for b in range(_NBUF):
            store_wait(b)

    return gather_kernel


def kernel(token_ids, weights):
    b, h = token_ids.shape
    return _make_gather(b, h, EMBEDDING_DIM)(token_ids.astype(jnp.int32), weights)


# speedup vs baseline: 1.0254x; 1.0254x over previous
import functools

import jax
import jax.numpy as jnp
from jax import lax
from jax.experimental import pallas as pl
from jax.experimental.pallas import tpu as pltpu
from jax.experimental.pallas import tpu_sc as plsc

EMBEDDING_DIM = 32
_NC = 2
_NS = 16
_NW = _NC * _NS
_NBUF = 2


@functools.lru_cache(maxsize=None)
def _make_gather(batch: int, hist: int, dim: int, rows_per_chunk: int):
    b_per_w = batch // _NW
    nchunks = b_per_w // rows_per_chunk
    chunk = rows_per_chunk * hist
    assert nchunks % _NBUF == 0
    hist_pad = (hist + 7) // 8 * 8
    dim_pad = 128
    mesh = plsc.VectorSubcoreMesh(core_axis_name="c", subcore_axis_name="s")

    scratch = (
        [pltpu.VMEM((chunk,), jnp.int32) for _ in range(_NBUF)]
        + [pltpu.VMEM((chunk, dim), jnp.float32) for _ in range(_NBUF)]
        + [pltpu.SemaphoreType.DMA for _ in range(3 * _NBUF)]
    )

    @functools.partial(
        pl.kernel,
        mesh=mesh,
        out_type=jax.ShapeDtypeStruct((batch, hist_pad, dim_pad), jnp.float32),
        scratch_types=scratch,
        compiler_params=pltpu.CompilerParams(use_tc_tiling_on_sc=False),
    )
    def gather_kernel(ids_hbm, table_hbm, out_hbm, *bufs):
        idx_v = bufs[0:_NBUF]
        rows_v = bufs[_NBUF:2 * _NBUF]
        sem_i = bufs[2 * _NBUF:3 * _NBUF]
        sem_r = bufs[3 * _NBUF:4 * _NBUF]
        sem_o = bufs[4 * _NBUF:5 * _NBUF]

        wid = lax.axis_index("s") * _NC + lax.axis_index("c")
        row_base = wid * b_per_w

        def ids_start(i, b):
            off = (row_base + i * rows_per_chunk) * hist
            pltpu.async_copy(ids_hbm.at[pl.ds(off, chunk)], idx_v[b], sem_i[b])

        def ids_wait(b):
            pltpu.make_async_copy(
                ids_hbm.at[pl.ds(0, chunk)], idx_v[b], sem_i[b]
            ).wait()

        def store_wait(b):
            for _ in range(rows_per_chunk):
                pltpu.make_async_copy(
                    rows_v[b].at[pl.ds(0, hist), :],
                    out_hbm.at[0, pl.ds(0, hist), pl.ds(0, dim)],
                    sem_o[b],
                ).wait()

        for b in range(_NBUF):
            ids_start(b, b)

        def group(g, carry):
            for b in range(_NBUF):
                i = g * _NBUF + b
                brow = row_base + i * rows_per_chunk

                ids_wait(b)

                @pl.when(g > 0)
                def _():
                    store_wait(b)

                pltpu.async_copy(table_hbm.at[idx_v[b]], rows_v[b], sem_r[b]).wait()

                for r in range(rows_per_chunk):
                    pltpu.async_copy(
                        rows_v[b].at[pl.ds(r * hist, hist), :],
                        out_hbm.at[brow + r, pl.ds(0, hist), pl.ds(0, dim)],
                        sem_o[b],
                    )

                @pl.when(i + _NBUF < nchunks)
                def _():
                    ids_start(i + _NBUF, b)

            return carry

        lax.fori_loop(0, nchunks // _NBUF, group, 0)

        for b in range(_NBUF):
            store_wait(b)

    return gather_kernel


def kernel(token_ids, weights):
    b, h = token_ids.shape
    ids_flat = token_ids.astype(jnp.int32).reshape(b * h)
    big = _make_gather(b, h, EMBEDDING_DIM, 32)(ids_flat, weights)
    one = jnp.float32(1) + (weights[0, 0] - weights[0, 0])
    return big[:, :h, :EMBEDDING_DIM] * one


# speedup vs baseline: 1.4306x; 1.3952x over previous
import functools

import jax
import jax.numpy as jnp
from jax import lax
from jax.experimental import pallas as pl
from jax.experimental.pallas import tpu as pltpu
from jax.experimental.pallas import tpu_sc as plsc

EMBEDDING_DIM = 32
_NC = 2
_NS = 16
_NW = _NC * _NS
_NBUF = 2


@functools.lru_cache(maxsize=None)
def _make_gather(batch: int, hist: int, dim: int, rows_per_chunk: int):
    b_per_w = batch // _NW
    nchunks = b_per_w // rows_per_chunk
    chunk = rows_per_chunk * hist
    assert nchunks % _NBUF == 0
    hist_pad = (hist + 7) // 8 * 8
    dim_pad = 128
    mesh = plsc.VectorSubcoreMesh(core_axis_name="c", subcore_axis_name="s")

    scratch = (
        [pltpu.VMEM((chunk,), jnp.int32) for _ in range(_NBUF)]
        + [pltpu.VMEM((chunk, dim), jnp.float32) for _ in range(_NBUF)]
        + [pltpu.SemaphoreType.DMA for _ in range(3 * _NBUF)]
    )

    @functools.partial(
        pl.kernel,
        mesh=mesh,
        out_type=jax.ShapeDtypeStruct((batch, hist_pad, dim_pad), jnp.float32),
        scratch_types=scratch,
        compiler_params=pltpu.CompilerParams(use_tc_tiling_on_sc=False),
    )
    def gather_kernel(ids_hbm, table_hbm, out_hbm, *bufs):
        idx_v = bufs[0:_NBUF]
        rows_v = bufs[_NBUF:2 * _NBUF]
        sem_i = bufs[2 * _NBUF:3 * _NBUF]
        sem_r = bufs[3 * _NBUF:4 * _NBUF]
        sem_o = bufs[4 * _NBUF:5 * _NBUF]

        wid = lax.axis_index("s") * _NC + lax.axis_index("c")
        row_base = wid * b_per_w

        def ids_start(i, b):
            off = (row_base + i * rows_per_chunk) * hist
            pltpu.async_copy(ids_hbm.at[pl.ds(off, chunk)], idx_v[b], sem_i[b])

        def ids_wait(b):
            pltpu.make_async_copy(
                ids_hbm.at[pl.ds(0, chunk)], idx_v[b], sem_i[b]
            ).wait()

        def store_wait(b):
            for _ in range(rows_per_chunk):
                pltpu.make_async_copy(
                    rows_v[b].at[pl.ds(0, hist), :],
                    out_hbm.at[0, pl.ds(0, hist), pl.ds(0, dim)],
                    sem_o[b],
                ).wait()

        for b in range(_NBUF):
            ids_start(b, b)

        def group(g, carry):
            for b in range(_NBUF):
                i = g * _NBUF + b
                brow = row_base + i * rows_per_chunk

                ids_wait(b)

                @pl.when(g > 0)
                def _():
                    store_wait(b)

                pltpu.async_copy(table_hbm.at[idx_v[b]], rows_v[b], sem_r[b]).wait()

                for r in range(rows_per_chunk):
                    pltpu.async_copy(
                        rows_v[b].at[pl.ds(r * hist, hist), :],
                        out_hbm.at[brow + r, pl.ds(0, hist), pl.ds(0, dim)],
                        sem_o[b],
                    )

                @pl.when(i + _NBUF < nchunks)
                def _():
                    ids_start(i + _NBUF, b)

            return carry

        lax.fori_loop(0, nchunks // _NBUF, group, 0)

        for b in range(_NBUF):
            store_wait(b)

    return gather_kernel


def kernel(token_ids, weights):
    b, h = token_ids.shape
    ids_flat = token_ids.astype(jnp.int32).reshape(b * h)
    big = _make_gather(b, h, EMBEDDING_DIM, 32)(ids_flat, weights)
    return big[:, :h, :EMBEDDING_DIM]
